# baseline (speedup 1.0000x reference)
import functools

import jax
import jax.numpy as jnp
from jax import lax
from jax.experimental import pallas as pl
from jax.experimental.pallas import tpu as pltpu
from jax.experimental.pallas import tpu_sc as plsc

DIM = 64
IDX_MINOR = 128
K = 5
NC = 2
NS = 16
NW = NC * NS
CHUNK = K * IDX_MINOR


@functools.lru_cache(maxsize=None)
def _gather_call(n_rows, dim):
    rows_per_w = n_rows // NW
    chunks_per_w = rows_per_w // CHUNK
    pairs = chunks_per_w // 2

    mesh = plsc.VectorSubcoreMesh(core_axis_name="c", subcore_axis_name="s")

    @functools.partial(
        pl.kernel,
        mesh=mesh,
        out_type=jax.ShapeDtypeStruct((n_rows, dim), jnp.float32),
        compiler_params=pltpu.CompilerParams(use_tc_tiling_on_sc=False),
        scratch_types=[
            pltpu.VMEM((K, IDX_MINOR), jnp.int32),
            pltpu.VMEM((K, IDX_MINOR), jnp.int32),
            pltpu.VMEM((CHUNK, dim), jnp.float32),
            pltpu.VMEM((CHUNK, dim), jnp.float32),
            pltpu.SemaphoreType.DMA,
            pltpu.SemaphoreType.DMA,
            pltpu.SemaphoreType.DMA,
        ],
    )
    def k(table_hbm, idx_hbm, out_hbm, idx0, idx1, buf0, buf1, g0s, g1s, wsem):
        wid = lax.axis_index("s") * NC + lax.axis_index("c")
        chunk0 = wid * chunks_per_w
        out_row0 = wid * rows_per_w

        def fire(cid, idx_v, buf, sem):
            pltpu.sync_copy(idx_hbm.at[cid], idx_v)
            for j in range(K):
                pltpu.async_copy(
                    table_hbm.at[idx_v.at[j]],
                    buf.at[pl.ds(j * IDX_MINOR, IDX_MINOR)],
                    sem,
                )

        def drain_gathers(buf, sem):
            for j in range(K):
                pltpu.make_async_copy(
                    table_hbm.at[pl.ds(0, IDX_MINOR)],
                    buf.at[pl.ds(j * IDX_MINOR, IDX_MINOR)],
                    sem,
                ).wait()

        def writeback(g, buf):
            return pltpu.async_copy(
                buf, out_hbm.at[pl.ds(out_row0 + g * CHUNK, CHUNK)], wsem
            )

        def drain_writebacks():
            pltpu.make_async_copy(
                buf0, out_hbm.at[pl.ds(out_row0, CHUNK)], wsem
            ).wait()
            pltpu.make_async_copy(
                buf1, out_hbm.at[pl.ds(out_row0, CHUNK)], wsem
            ).wait()

        def body(t, carry):
            g0 = 2 * t
            g1 = g0 + 1

            @pl.when(t > 0)
            def _():
                drain_writebacks()

            fire(chunk0 + g0, idx0, buf0, g0s)
            fire(chunk0 + g1, idx1, buf1, g1s)
            drain_gathers(buf0, g0s)
            writeback(g0, buf0)
            drain_gathers(buf1, g1s)
            writeback(g1, buf1)
            return carry

        lax.fori_loop(0, pairs, body, 0)
        drain_writebacks()

    return k


def kernel(x, table):
    b, l = x.shape
    n = b * l
    idx3d = x.reshape(n // CHUNK, K, IDX_MINOR).astype(jnp.int32)
    out = _gather_call(n, table.shape[1])(table, idx3d)
    return out.reshape(b, l, table.shape[1])
